# baseline (speedup 1.0000x reference)
import functools

import jax
import jax.numpy as jnp
from jax import lax
from jax.experimental import pallas as pl
from jax.experimental.pallas import tpu as pltpu
from jax.experimental.pallas import tpu_sc as plsc

_NUM_CORES = 2
_NUM_SUBCORES = 16
_NW = _NUM_CORES * _NUM_SUBCORES
_CHUNK = 128


@functools.lru_cache(maxsize=None)
def _build(n_rows, dim):
    rows_per_w = n_rows // _NW
    chunks_per_w = rows_per_w // _CHUNK
    mesh = plsc.VectorSubcoreMesh(core_axis_name="c", subcore_axis_name="s")

    @functools.partial(
        pl.kernel,
        mesh=mesh,
        out_type=jax.ShapeDtypeStruct((n_rows, dim), jnp.float32),
        scratch_types=[
            pltpu.VMEM((chunks_per_w, _CHUNK), jnp.int32),
            pltpu.VMEM((_CHUNK, dim), jnp.float32),
            pltpu.SemaphoreType.DMA,
        ],
        compiler_params=pltpu.CompilerParams(use_tc_tiling_on_sc=False),
    )
    def run(idx_hbm, table_hbm, out_hbm, idx_v, rows_v, sem):
        wid = lax.axis_index("s") * _NUM_CORES + lax.axis_index("c")
        pltpu.sync_copy(idx_hbm.at[pl.ds(wid * chunks_per_w, chunks_per_w)], idx_v)
        base = wid * rows_per_w

        def body(j, carry):
            pltpu.async_copy(table_hbm.at[idx_v.at[j]], rows_v, sem).wait()
            pltpu.sync_copy(rows_v, out_hbm.at[pl.ds(base + j * _CHUNK, _CHUNK)])
            return carry

        lax.fori_loop(0, chunks_per_w, body, 0)

    return run


def kernel(token_ids, weight):
    n_rows = token_ids.size
    dim = weight.shape[1]
    idx = token_ids.reshape(n_rows // _CHUNK, _CHUNK).astype(jnp.int32)
    out = _build(n_rows, dim)(idx, weight)
    return out.reshape(token_ids.shape + (dim,))


# speedup vs baseline: 1.1120x; 1.1120x over previous
import functools

import jax
import jax.numpy as jnp
from jax import lax
from jax.experimental import pallas as pl
from jax.experimental.pallas import tpu as pltpu
from jax.experimental.pallas import tpu_sc as plsc

_NUM_CORES = 2
_NUM_SUBCORES = 16
_NW = _NUM_CORES * _NUM_SUBCORES
_CHUNK = 128
_NBUF = 4


@functools.lru_cache(maxsize=None)
def _build(n_rows, dim):
    rows_per_w = n_rows // _NW
    chunks_per_w = rows_per_w // _CHUNK
    n_groups = chunks_per_w // _NBUF
    mesh = plsc.VectorSubcoreMesh(core_axis_name="c", subcore_axis_name="s")

    @functools.partial(
        pl.kernel,
        mesh=mesh,
        out_type=jax.ShapeDtypeStruct((n_rows, dim), jnp.float32),
        scratch_types=(
            [pltpu.VMEM((chunks_per_w, _CHUNK), jnp.int32)]
            + [pltpu.VMEM((_CHUNK, dim), jnp.float32) for _ in range(_NBUF)]
            + [pltpu.SemaphoreType.DMA for _ in range(2 * _NBUF)]
        ),
        compiler_params=pltpu.CompilerParams(
            use_tc_tiling_on_sc=False, skip_device_barrier=True
        ),
    )
    def run(idx_hbm, table_hbm, out_hbm, idx_v, *bufs_and_sems):
        bufs = bufs_and_sems[:_NBUF]
        gsems = bufs_and_sems[_NBUF : 2 * _NBUF]
        osems = bufs_and_sems[2 * _NBUF :]
        wid = lax.axis_index("s") * _NUM_CORES + lax.axis_index("c")
        pltpu.sync_copy(idx_hbm.at[pl.ds(wid * chunks_per_w, chunks_per_w)], idx_v)
        base = wid * rows_per_w

        def gather(j, b):
            return pltpu.make_async_copy(table_hbm.at[idx_v.at[j]], bufs[b], gsems[b])

        def store(j, b):
            return pltpu.make_async_copy(
                bufs[b], out_hbm.at[pl.ds(base + j * _CHUNK, _CHUNK)], osems[b]
            )

        for b in range(_NBUF):
            gather(b, b).start()

        def loop_body(g, carry):
            j0 = g * _NBUF
            for b in range(_NBUF):
                gather(j0 + b, b).wait()
                store(j0 + b, b).start()
            for b in range(_NBUF):
                store(j0 + b, b).wait()
                nj = j0 + b + _NBUF

                @pl.when(nj < chunks_per_w)
                def _():
                    gather(nj, b).start()

            return carry

        lax.fori_loop(0, n_groups, loop_body, 0)

    return run


def kernel(token_ids, weight):
    n_rows = token_ids.size
    dim = weight.shape[1]
    idx = token_ids.reshape(n_rows // _CHUNK, _CHUNK).astype(jnp.int32)
    out = _build(n_rows, dim)(idx, weight)
    return out.reshape(token_ids.shape + (dim,))
